# Initial kernel scaffold; baseline (speedup 1.0000x reference)
import jax
import jax.numpy as jnp
from jax.experimental import pallas as pl


def kernel(x, edge_index, batch, W1, b1, g1, be1, W2, b2, g2, be2, W3, b3):
    raise NotImplementedError("write your pallas kernel here")


# speedup vs baseline: 15.6354x; 15.6354x over previous
import functools

import jax
import jax.numpy as jnp
from jax import lax
from jax.experimental import pallas as pl
from jax.experimental.pallas import tpu as pltpu
from jax.experimental.pallas import tpu_sc as plsc

_N = 10000
_E = 320000
_G = 64
_C = 10

_NC = 2
_NS = 16
_NW = _NC * _NS
_EPW = _E // _NW
_CHUNK = 80
_NCH = _EPW // _CHUNK
_NP = 10240
_RPT = _NP // _NS
_ZR = 32


def _scatter_sc(h):
    mesh = plsc.VectorSubcoreMesh(core_axis_name="c", subcore_axis_name="s")

    @functools.partial(
        pl.kernel,
        mesh=mesh,
        out_type=jax.ShapeDtypeStruct((_NC, _NP, h), jnp.float32),
        scratch_types=[
            pltpu.VMEM_SHARED((_NP, h), jnp.float32),
            pltpu.VMEM((_NCH, _CHUNK), jnp.int32),
            pltpu.VMEM((_NCH, _CHUNK), jnp.int32),
            pltpu.VMEM((_CHUNK, h), jnp.float32),
            pltpu.VMEM((_ZR, h), jnp.float32),
            pltpu.SemaphoreType.DMA,
        ],
    )
    def scat(y_hbm, src_hbm, dst_hbm, out_hbm, acc_sh, src_v, dst_v,
             rows_v, zbuf_v, sem):
        c = lax.axis_index("c")
        s = lax.axis_index("s")
        wid = c * _NS + s

        pltpu.sync_copy(src_hbm.at[wid], src_v)
        pltpu.sync_copy(dst_hbm.at[wid], dst_v)

        zv = jnp.zeros((16,), jnp.float32)

        def zrow(i, carry):
            def zcol(k, carry2):
                zbuf_v[i, pl.ds(pl.multiple_of(k * 16, 16), 16)] = zv
                return carry2
            return lax.fori_loop(0, h // 16, zcol, carry)

        lax.fori_loop(0, _ZR, zrow, 0)
        base = s * _RPT
        for k in range(_RPT // _ZR):
            pltpu.sync_copy(zbuf_v, acc_sh.at[pl.ds(base + k * _ZR, _ZR)])
        plsc.subcore_barrier()

        def step(j, carry):
            pltpu.async_copy(y_hbm.at[src_v.at[j]], rows_v, sem).wait()
            pltpu.sync_copy(rows_v, acc_sh.at[dst_v.at[j]], add=True)
            return carry

        lax.fori_loop(0, _NCH, step, 0)
        plsc.subcore_barrier()

        for k in range(_RPT // _ZR):
            row0 = base + k * _ZR
            pltpu.sync_copy(acc_sh.at[pl.ds(row0, _ZR)], zbuf_v)
            pltpu.sync_copy(zbuf_v, out_hbm.at[c, pl.ds(row0, _ZR)])

    return scat


def _degree_sc():
    mesh = plsc.VectorSubcoreMesh(core_axis_name="c", subcore_axis_name="s")
    h = 128

    @functools.partial(
        pl.kernel,
        mesh=mesh,
        out_type=jax.ShapeDtypeStruct((_NC, _NP, h), jnp.float32),
        scratch_types=[
            pltpu.VMEM_SHARED((_NP, h), jnp.float32),
            pltpu.VMEM((_NCH, _CHUNK), jnp.int32),
            pltpu.VMEM((_CHUNK, h), jnp.float32),
            pltpu.VMEM((_ZR, h), jnp.float32),
        ],
    )
    def degk(dst_hbm, out_hbm, acc_sh, dst_v, ones_v, zbuf_v):
        c = lax.axis_index("c")
        s = lax.axis_index("s")
        wid = c * _NS + s

        pltpu.sync_copy(dst_hbm.at[wid], dst_v)

        zv = jnp.zeros((16,), jnp.float32)
        ov = jnp.ones((16,), jnp.float32)

        def frow(i, carry):
            def fcol(k, carry2):
                zbuf_v[i, pl.ds(pl.multiple_of(k * 16, 16), 16)] = zv
                return carry2
            return lax.fori_loop(0, h // 16, fcol, carry)

        def orow(i, carry):
            def ocol(k, carry2):
                ones_v[i, pl.ds(pl.multiple_of(k * 16, 16), 16)] = ov
                return carry2
            return lax.fori_loop(0, h // 16, ocol, carry)

        lax.fori_loop(0, _ZR, frow, 0)
        lax.fori_loop(0, _CHUNK, orow, 0)
        base = s * _RPT
        for k in range(_RPT // _ZR):
            pltpu.sync_copy(zbuf_v, acc_sh.at[pl.ds(base + k * _ZR, _ZR)])
        plsc.subcore_barrier()

        def step(j, carry):
            pltpu.sync_copy(ones_v, acc_sh.at[dst_v.at[j]], add=True)
            return carry

        lax.fori_loop(0, _NCH, step, 0)
        plsc.subcore_barrier()

        for k in range(_RPT // _ZR):
            row0 = base + k * _ZR
            pltpu.sync_copy(acc_sh.at[pl.ds(row0, _ZR)], zbuf_v)
            pltpu.sync_copy(zbuf_v, out_hbm.at[c, pl.ds(row0, _ZR)])

    return degk


def _k1_body(deg_ref, x_ref, w_ref, y_ref, dis_ref):
    deg = deg_ref[0, :_N, 0:1] + deg_ref[1, :_N, 0:1] + 1.0
    dis = lax.rsqrt(jnp.maximum(deg, 1e-12))
    xw = jnp.dot(x_ref[...], w_ref[...], preferred_element_type=jnp.float32)
    y_ref[...] = xw * dis
    dis_ref[...] = dis


def _k2_body(hin, a_ref, y_ref, dis_ref, g_ref, be_ref, w_ref, o_ref):
    dis = dis_ref[...]
    v = (a_ref[0, :_N, :hin] + a_ref[1, :_N, :hin]
         + y_ref[..., :hin]) * dis
    mu = jnp.mean(v, axis=0, keepdims=True)
    var = jnp.mean((v - mu) * (v - mu), axis=0, keepdims=True)
    hbn = (v - mu) * lax.rsqrt(var + 1e-5) * g_ref[...] + be_ref[...]
    hr = jnp.maximum(hbn, 0.0)
    o_ref[...] = jnp.dot(hr, w_ref[...],
                         preferred_element_type=jnp.float32) * dis


def _k3_body(a_ref, y_ref, dis_ref, b_ref, batch_ref, o_ref):
    hfin = (a_ref[0, :_N] + a_ref[1, :_N] + y_ref[...]) * dis_ref[...] + b_ref[...]
    seg = lax.broadcasted_iota(jnp.int32, (_G, _N), 0)
    p = jnp.where(batch_ref[...] == seg, 1.0, 0.0)
    sums = jnp.dot(p, hfin, preferred_element_type=jnp.float32)
    cnt = jnp.sum(p, axis=1, keepdims=True)
    pooled = sums[:, :_C] / jnp.maximum(cnt, 1.0)
    m = jnp.max(pooled, axis=1, keepdims=True)
    ex = jnp.exp(pooled - m)
    lse = jnp.log(jnp.sum(ex, axis=1, keepdims=True)) + m
    o_ref[...] = pooled - lse


def _tc_call(body, out_shapes, *args):
    return pl.pallas_call(
        body,
        out_shape=out_shapes,
    )(*args)


def kernel(x, edge_index, batch, W1, b1, g1, be1, W2, b2, g2, be2, W3, b3):
    src = edge_index[0].reshape(_NW, _NCH, _CHUNK)
    dst = edge_index[1].reshape(_NW, _NCH, _CHUNK)
    batch2 = batch.reshape(1, _N)
    w1p = jnp.pad(W1, ((0, 0), (0, 64)))
    w3p = jnp.pad(W3, ((0, 0), (0, 128 - _C)))
    b3p = jnp.pad(b3, (0, 128 - _C)).reshape(1, 128)
    g1r = g1.reshape(1, -1)
    be1r = be1.reshape(1, -1)
    g2r = g2.reshape(1, -1)
    be2r = be2.reshape(1, -1)

    deg2 = _degree_sc()(dst)

    y1, dis = _tc_call(
        _k1_body,
        (jax.ShapeDtypeStruct((_N, 128), jnp.float32),
         jax.ShapeDtypeStruct((_N, 1), jnp.float32)),
        deg2, x, w1p)

    a1 = _scatter_sc(128)(y1, src, dst)
    y2 = _tc_call(
        functools.partial(_k2_body, 64),
        jax.ShapeDtypeStruct((_N, 128), jnp.float32),
        a1, y1, dis, g1r, be1r, W2)

    a2 = _scatter_sc(128)(y2, src, dst)
    y3 = _tc_call(
        functools.partial(_k2_body, 128),
        jax.ShapeDtypeStruct((_N, 128), jnp.float32),
        a2, y2, dis, g2r, be2r, w3p)

    a3 = _scatter_sc(128)(y3, src, dst)
    out = _tc_call(
        _k3_body,
        jax.ShapeDtypeStruct((_G, _C), jnp.float32),
        a3, y3, dis, b3p, batch2)
    return out


# speedup vs baseline: 24.4364x; 1.5629x over previous
import functools

import jax
import jax.numpy as jnp
from jax import lax
from jax.experimental import pallas as pl
from jax.experimental.pallas import tpu as pltpu
from jax.experimental.pallas import tpu_sc as plsc

_N = 10000
_E = 320000
_G = 64
_C = 10

_NC = 2
_NS = 16
_NW = _NC * _NS
_EPW = _E // _NW
_CHUNK = 128
_NCH = 80
_EPAD = _NCH * _CHUNK - _EPW
_GC = 10
_NG = _NCH // _GC
_DW = 128
_NP = 10240
_RPT = _NP // _NS
_ZR = 32


def _scatter_sc(h):
    mesh = plsc.VectorSubcoreMesh(core_axis_name="c", subcore_axis_name="s")

    @functools.partial(
        pl.kernel,
        mesh=mesh,
        out_type=jax.ShapeDtypeStruct((_NC, _NP, h), jnp.float32),
        scratch_types=[
            pltpu.VMEM_SHARED((_NP, h), jnp.float32),
            pltpu.VMEM((_GC, 2, _CHUNK), jnp.int32),
            pltpu.VMEM((_GC, 2, _CHUNK), jnp.int32),
            pltpu.VMEM((_CHUNK, h), jnp.float32),
            pltpu.VMEM((_CHUNK, h), jnp.float32),
            pltpu.SemaphoreType.DMA,
            pltpu.SemaphoreType.DMA,
            pltpu.SemaphoreType.DMA,
        ],
    )
    def scat(y_hbm, e_hbm, out_hbm, acc_sh, ga, gb, rb0, rb1,
             gs0, gs1, isem):
        c = lax.axis_index("c")
        s = lax.axis_index("s")
        wid = c * _NS + s
        base = s * _RPT

        zv = jnp.zeros((16,), jnp.float32)

        def zrow(i, carry):
            def zcol(k, carry2):
                rb0[i, pl.ds(pl.multiple_of(k * 16, 16), 16)] = zv
                return carry2
            return lax.fori_loop(0, h // 16, zcol, carry)

        lax.fori_loop(0, _CHUNK, zrow, 0)
        for k in range(_RPT // _CHUNK):
            pltpu.sync_copy(rb0, acc_sh.at[pl.ds(base + k * _CHUNK, _CHUNK)])
        plsc.subcore_barrier()

        rbs = [rb0, rb1]
        sems = [gs0, gs1]

        def group(gbuf):
            hnd = [None] * _GC
            for j in range(2):
                hnd[j] = pltpu.async_copy(
                    y_hbm.at[gbuf.at[j, 0]], rbs[j % 2], sems[j % 2])
            for j in range(_GC):
                hnd[j].wait()
                pltpu.sync_copy(rbs[j % 2], acc_sh.at[gbuf.at[j, 1]],
                                add=True)
                if j + 2 < _GC:
                    hnd[j + 2] = pltpu.async_copy(
                        y_hbm.at[gbuf.at[j + 2, 0]], rbs[j % 2],
                        sems[j % 2])

        pltpu.sync_copy(e_hbm.at[wid, pl.ds(0, _GC)], ga)

        def pair(k, carry):
            g = 2 * k
            hb = pltpu.async_copy(
                e_hbm.at[wid, pl.ds(jnp.minimum(g + 1, _NG - 1) * _GC, _GC)],
                gb, isem)
            group(ga)
            hb.wait()
            ha = pltpu.async_copy(
                e_hbm.at[wid, pl.ds(jnp.minimum(g + 2, _NG - 1) * _GC, _GC)],
                ga, isem)
            group(gb)
            ha.wait()
            return carry

        lax.fori_loop(0, _NG // 2, pair, 0)
        plsc.subcore_barrier()

        for k in range(_RPT // _CHUNK):
            row0 = base + k * _CHUNK
            pltpu.sync_copy(acc_sh.at[pl.ds(row0, _CHUNK)], rb0)
            pltpu.sync_copy(rb0, out_hbm.at[c, pl.ds(row0, _CHUNK)])

    return scat


def _degree_sc():
    mesh = plsc.VectorSubcoreMesh(core_axis_name="c", subcore_axis_name="s")
    h = _DW

    @functools.partial(
        pl.kernel,
        mesh=mesh,
        out_type=jax.ShapeDtypeStruct((_NC, _NP, h), jnp.float32),
        scratch_types=[
            pltpu.VMEM_SHARED((_NP, h), jnp.float32),
            pltpu.VMEM((_NCH, _CHUNK), jnp.int32),
            pltpu.VMEM((_CHUNK, h), jnp.float32),
            pltpu.VMEM((_CHUNK, h), jnp.float32),
        ],
    )
    def degk(dst_hbm, out_hbm, acc_sh, dst_v, ones_v, zbuf_v):
        c = lax.axis_index("c")
        s = lax.axis_index("s")
        wid = c * _NS + s
        base = s * _RPT

        pltpu.sync_copy(dst_hbm.at[wid], dst_v)

        zv = jnp.zeros((16,), jnp.float32)
        ov = jnp.ones((16,), jnp.float32)

        def fill(i, carry):
            def fcol(k, carry2):
                kk = pl.ds(pl.multiple_of(k * 16, 16), 16)
                zbuf_v[i, kk] = zv
                ones_v[i, kk] = ov
                return carry2
            return lax.fori_loop(0, h // 16, fcol, carry)

        lax.fori_loop(0, _CHUNK, fill, 0)
        for k in range(_RPT // _CHUNK):
            pltpu.sync_copy(zbuf_v, acc_sh.at[pl.ds(base + k * _CHUNK, _CHUNK)])
        plsc.subcore_barrier()

        def step(j, carry):
            pltpu.sync_copy(ones_v, acc_sh.at[dst_v.at[j]], add=True)
            return carry

        lax.fori_loop(0, _NCH, step, 0)
        plsc.subcore_barrier()

        for k in range(_RPT // _CHUNK):
            row0 = base + k * _CHUNK
            pltpu.sync_copy(acc_sh.at[pl.ds(row0, _CHUNK)], zbuf_v)
            pltpu.sync_copy(zbuf_v, out_hbm.at[c, pl.ds(row0, _CHUNK)])

    return degk


def _k1_body(deg_ref, x_ref, w_ref, y_ref, dis_ref):
    deg = deg_ref[0, :_N, 0:1] + deg_ref[1, :_N, 0:1] + 1.0
    dis = lax.rsqrt(jnp.maximum(deg, 1e-12))
    xw = jnp.dot(x_ref[...], w_ref[...], preferred_element_type=jnp.float32)
    y_ref[...] = xw * dis
    dis_ref[...] = dis


def _k2_body(hin, a_ref, y_ref, dis_ref, g_ref, be_ref, w_ref, o_ref):
    dis = dis_ref[...]
    v = (a_ref[0, :_N, :hin] + a_ref[1, :_N, :hin]
         + y_ref[..., :hin]) * dis
    mu = jnp.mean(v, axis=0, keepdims=True)
    var = jnp.mean((v - mu) * (v - mu), axis=0, keepdims=True)
    hbn = (v - mu) * lax.rsqrt(var + 1e-5) * g_ref[...] + be_ref[...]
    hr = jnp.maximum(hbn, 0.0)
    o_ref[...] = jnp.dot(hr, w_ref[...],
                         preferred_element_type=jnp.float32) * dis


def _k3_body(a_ref, y_ref, dis_ref, b_ref, batch_ref, o_ref):
    hfin = (a_ref[0, :_N] + a_ref[1, :_N] + y_ref[...]) * dis_ref[...] + b_ref[...]
    seg = lax.broadcasted_iota(jnp.int32, (_G, _N), 0)
    p = jnp.where(batch_ref[...] == seg, 1.0, 0.0)
    sums = jnp.dot(p, hfin, preferred_element_type=jnp.float32)
    cnt = jnp.sum(p, axis=1, keepdims=True)
    pooled = sums[:, :_C] / jnp.maximum(cnt, 1.0)
    m = jnp.max(pooled, axis=1, keepdims=True)
    ex = jnp.exp(pooled - m)
    lse = jnp.log(jnp.sum(ex, axis=1, keepdims=True)) + m
    o_ref[...] = pooled - lse


def _tc_call(body, out_shapes, *args):
    return pl.pallas_call(
        body,
        out_shape=out_shapes,
    )(*args)


def kernel(x, edge_index, batch, W1, b1, g1, be1, W2, b2, g2, be2, W3, b3):
    dum_src = jnp.broadcast_to(
        jnp.arange(_EPAD, dtype=jnp.int32)[None, :], (_NW, _EPAD))
    dum_dst = jnp.broadcast_to(
        (_N + jnp.arange(_EPAD, dtype=jnp.int32) % (_NP - _N))[None, :],
        (_NW, _EPAD))
    srcp = jnp.concatenate(
        [edge_index[0].reshape(_NW, _EPW), dum_src], axis=1)
    dstp = jnp.concatenate(
        [edge_index[1].reshape(_NW, _EPW), dum_dst], axis=1)
    edges = jnp.stack([srcp.reshape(_NW, _NCH, _CHUNK),
                       dstp.reshape(_NW, _NCH, _CHUNK)], axis=2)
    dst = dstp.reshape(_NW, _NCH, _CHUNK)
    batch2 = batch.reshape(1, _N)
    w1p = jnp.pad(W1, ((0, 0), (0, 64)))
    w3p = jnp.pad(W3, ((0, 0), (0, 128 - _C)))
    b3p = jnp.pad(b3, (0, 128 - _C)).reshape(1, 128)
    g1r = g1.reshape(1, -1)
    be1r = be1.reshape(1, -1)
    g2r = g2.reshape(1, -1)
    be2r = be2.reshape(1, -1)

    deg2 = _degree_sc()(dst)

    y1, dis = _tc_call(
        _k1_body,
        (jax.ShapeDtypeStruct((_N, 128), jnp.float32),
         jax.ShapeDtypeStruct((_N, 1), jnp.float32)),
        deg2, x, w1p)

    a1 = _scatter_sc(128)(y1, edges)
    y2 = _tc_call(
        functools.partial(_k2_body, 64),
        jax.ShapeDtypeStruct((_N, 128), jnp.float32),
        a1, y1, dis, g1r, be1r, W2)

    a2 = _scatter_sc(128)(y2, edges)
    y3 = _tc_call(
        functools.partial(_k2_body, 128),
        jax.ShapeDtypeStruct((_N, 128), jnp.float32),
        a2, y2, dis, g2r, be2r, w3p)

    a3 = _scatter_sc(128)(y3, edges)
    out = _tc_call(
        _k3_body,
        jax.ShapeDtypeStruct((_G, _C), jnp.float32),
        a3, y3, dis, b3p, batch2)
    return out


# speedup vs baseline: 24.4685x; 1.0013x over previous
import functools

import jax
import jax.numpy as jnp
from jax import lax
from jax.experimental import pallas as pl
from jax.experimental.pallas import tpu as pltpu
from jax.experimental.pallas import tpu_sc as plsc

_N = 10000
_E = 320000
_G = 64
_C = 10

_NC = 2
_NS = 16
_NW = _NC * _NS
_EPW = _E // _NW
_CHUNK = 128
_NCH = 80
_EPAD = _NCH * _CHUNK - _EPW
_GC = 10
_NG = _NCH // _GC
_DW = 128
_NP = 10240
_RPT = _NP // _NS
_ZR = 32


def _scatter_sc(h):
    mesh = plsc.VectorSubcoreMesh(core_axis_name="c", subcore_axis_name="s")

    @functools.partial(
        pl.kernel,
        mesh=mesh,
        out_type=jax.ShapeDtypeStruct((_NC, _NP, h), jnp.float32),
        scratch_types=[
            pltpu.VMEM_SHARED((_NP, h), jnp.float32),
            pltpu.VMEM((_GC, 2, _CHUNK), jnp.int32),
            pltpu.VMEM((_GC, 2, _CHUNK), jnp.int32),
            pltpu.VMEM((_CHUNK, h), jnp.float32),
            pltpu.VMEM((_CHUNK, h), jnp.float32),
            pltpu.SemaphoreType.DMA,
            pltpu.SemaphoreType.DMA,
            pltpu.SemaphoreType.DMA,
        ],
    )
    def scat(y_hbm, e_hbm, out_hbm, acc_sh, ga, gb, rb0, rb1,
             gs0, gs1, isem):
        c = lax.axis_index("c")
        s = lax.axis_index("s")
        wid = c * _NS + s
        base = s * _RPT

        zv = jnp.zeros((16,), jnp.float32)

        def zrow(i, carry):
            def zcol(k, carry2):
                rb0[i, pl.ds(pl.multiple_of(k * 16, 16), 16)] = zv
                return carry2
            return lax.fori_loop(0, h // 16, zcol, carry)

        lax.fori_loop(0, _CHUNK, zrow, 0)
        for k in range(_RPT // _CHUNK):
            pltpu.sync_copy(rb0, acc_sh.at[pl.ds(base + k * _CHUNK, _CHUNK)])
        plsc.subcore_barrier()

        rbs = [rb0, rb1]
        sems = [gs0, gs1]

        def group(gbuf):
            hnd = [None] * _GC
            for j in range(2):
                hnd[j] = pltpu.async_copy(
                    y_hbm.at[gbuf.at[j, 0]], rbs[j % 2], sems[j % 2])
            for j in range(_GC):
                hnd[j].wait()
                pltpu.sync_copy(rbs[j % 2], acc_sh.at[gbuf.at[j, 1]],
                                add=True)
                if j + 2 < _GC:
                    hnd[j + 2] = pltpu.async_copy(
                        y_hbm.at[gbuf.at[j + 2, 0]], rbs[j % 2],
                        sems[j % 2])

        pltpu.sync_copy(e_hbm.at[wid, pl.ds(0, _GC)], ga)

        def pair(k, carry):
            g = 2 * k
            hb = pltpu.async_copy(
                e_hbm.at[wid, pl.ds(jnp.minimum(g + 1, _NG - 1) * _GC, _GC)],
                gb, isem)
            group(ga)
            hb.wait()
            ha = pltpu.async_copy(
                e_hbm.at[wid, pl.ds(jnp.minimum(g + 2, _NG - 1) * _GC, _GC)],
                ga, isem)
            group(gb)
            ha.wait()
            return carry

        lax.fori_loop(0, _NG // 2, pair, 0)
        plsc.subcore_barrier()

        pltpu.sync_copy(acc_sh.at[pl.ds(base, _RPT)],
                        out_hbm.at[c, pl.ds(base, _RPT)])

    return scat


def _degree_sc():
    mesh = plsc.VectorSubcoreMesh(core_axis_name="c", subcore_axis_name="s")
    h = _DW

    @functools.partial(
        pl.kernel,
        mesh=mesh,
        out_type=jax.ShapeDtypeStruct((_NC, _NP, h), jnp.float32),
        scratch_types=[
            pltpu.VMEM_SHARED((_NP, h), jnp.float32),
            pltpu.VMEM((_NCH, _CHUNK), jnp.int32),
            pltpu.VMEM((_CHUNK, h), jnp.float32),
            pltpu.VMEM((_CHUNK, h), jnp.float32),
        ],
    )
    def degk(dst_hbm, out_hbm, acc_sh, dst_v, ones_v, zbuf_v):
        c = lax.axis_index("c")
        s = lax.axis_index("s")
        wid = c * _NS + s
        base = s * _RPT

        pltpu.sync_copy(dst_hbm.at[wid], dst_v)

        zv = jnp.zeros((16,), jnp.float32)
        ov = jnp.ones((16,), jnp.float32)

        def fill(i, carry):
            def fcol(k, carry2):
                kk = pl.ds(pl.multiple_of(k * 16, 16), 16)
                zbuf_v[i, kk] = zv
                ones_v[i, kk] = ov
                return carry2
            return lax.fori_loop(0, h // 16, fcol, carry)

        lax.fori_loop(0, _CHUNK, fill, 0)
        for k in range(_RPT // _CHUNK):
            pltpu.sync_copy(zbuf_v, acc_sh.at[pl.ds(base + k * _CHUNK, _CHUNK)])
        plsc.subcore_barrier()

        def step(j, carry):
            pltpu.sync_copy(ones_v, acc_sh.at[dst_v.at[j]], add=True)
            return carry

        lax.fori_loop(0, _NCH, step, 0)
        plsc.subcore_barrier()

        pltpu.sync_copy(acc_sh.at[pl.ds(base, _RPT)],
                        out_hbm.at[c, pl.ds(base, _RPT)])

    return degk


def _mm_body(x_ref, w_ref, o_ref):
    o_ref[...] = jnp.dot(x_ref[...], w_ref[...],
                         preferred_element_type=jnp.float32)


def _k1_body(deg_ref, xw_ref, y_ref, dis_ref):
    deg = deg_ref[0, :_N, 0:1] + deg_ref[1, :_N, 0:1] + 1.0
    dis = lax.rsqrt(jnp.maximum(deg, 1e-12))
    y_ref[...] = xw_ref[...] * dis
    dis_ref[...] = dis


def _k2_body(hin, a_ref, y_ref, dis_ref, g_ref, be_ref, w_ref, o_ref):
    dis = dis_ref[...]
    v = (a_ref[0, :_N, :hin] + a_ref[1, :_N, :hin]
         + y_ref[..., :hin]) * dis
    mu = jnp.mean(v, axis=0, keepdims=True)
    var = jnp.mean((v - mu) * (v - mu), axis=0, keepdims=True)
    hbn = (v - mu) * lax.rsqrt(var + 1e-5) * g_ref[...] + be_ref[...]
    hr = jnp.maximum(hbn, 0.0)
    o_ref[...] = jnp.dot(hr, w_ref[...],
                         preferred_element_type=jnp.float32) * dis


def _k3_body(a_ref, y_ref, dis_ref, b_ref, batch_ref, o_ref):
    hfin = (a_ref[0, :_N] + a_ref[1, :_N] + y_ref[...]) * dis_ref[...] + b_ref[...]
    seg = lax.broadcasted_iota(jnp.int32, (_G, _N), 0)
    p = jnp.where(batch_ref[...] == seg, 1.0, 0.0)
    sums = jnp.dot(p, hfin, preferred_element_type=jnp.float32)
    cnt = jnp.sum(p, axis=1, keepdims=True)
    pooled = sums[:, :_C] / jnp.maximum(cnt, 1.0)
    m = jnp.max(pooled, axis=1, keepdims=True)
    ex = jnp.exp(pooled - m)
    lse = jnp.log(jnp.sum(ex, axis=1, keepdims=True)) + m
    o_ref[...] = pooled - lse


def _tc_call(body, out_shapes, *args):
    return pl.pallas_call(
        body,
        out_shape=out_shapes,
    )(*args)


def kernel(x, edge_index, batch, W1, b1, g1, be1, W2, b2, g2, be2, W3, b3):
    dum_src = jnp.broadcast_to(
        jnp.arange(_EPAD, dtype=jnp.int32)[None, :], (_NW, _EPAD))
    dum_dst = jnp.broadcast_to(
        (_N + jnp.arange(_EPAD, dtype=jnp.int32) % (_NP - _N))[None, :],
        (_NW, _EPAD))
    srcp = jnp.concatenate(
        [edge_index[0].reshape(_NW, _EPW), dum_src], axis=1)
    dstp = jnp.concatenate(
        [edge_index[1].reshape(_NW, _EPW), dum_dst], axis=1)
    edges = jnp.stack([srcp.reshape(_NW, _NCH, _CHUNK),
                       dstp.reshape(_NW, _NCH, _CHUNK)], axis=2)
    dst = dstp.reshape(_NW, _NCH, _CHUNK)
    batch2 = batch.reshape(1, _N)
    w1p = jnp.pad(W1, ((0, 0), (0, 64)))
    w3p = jnp.pad(W3, ((0, 0), (0, 128 - _C)))
    b3p = jnp.pad(b3, (0, 128 - _C)).reshape(1, 128)
    g1r = g1.reshape(1, -1)
    be1r = be1.reshape(1, -1)
    g2r = g2.reshape(1, -1)
    be2r = be2.reshape(1, -1)

    deg2 = _degree_sc()(dst)
    xw1 = _tc_call(
        _mm_body,
        jax.ShapeDtypeStruct((_N, 128), jnp.float32),
        x, w1p)

    y1, dis = _tc_call(
        _k1_body,
        (jax.ShapeDtypeStruct((_N, 128), jnp.float32),
         jax.ShapeDtypeStruct((_N, 1), jnp.float32)),
        deg2, xw1)

    a1 = _scatter_sc(128)(y1, edges)
    y2 = _tc_call(
        functools.partial(_k2_body, 64),
        jax.ShapeDtypeStruct((_N, 128), jnp.float32),
        a1, y1, dis, g1r, be1r, W2)

    a2 = _scatter_sc(128)(y2, edges)
    y3 = _tc_call(
        functools.partial(_k2_body, 128),
        jax.ShapeDtypeStruct((_N, 128), jnp.float32),
        a2, y2, dis, g2r, be2r, w3p)

    a3 = _scatter_sc(128)(y3, edges)
    out = _tc_call(
        _k3_body,
        jax.ShapeDtypeStruct((_G, _C), jnp.float32),
        a3, y3, dis, b3p, batch2)
    return out


# speedup vs baseline: 29.9008x; 1.2220x over previous
import functools

import jax
import jax.numpy as jnp
from jax import lax
from jax.experimental import pallas as pl
from jax.experimental.pallas import tpu as pltpu
from jax.experimental.pallas import tpu_sc as plsc

_N = 10000
_E = 320000
_G = 64
_C = 10

_NC = 2
_NS = 16
_NW = _NC * _NS
_EPW = _E // _NW
_CHUNK = 128
_NCH = 80
_EPAD = _NCH * _CHUNK - _EPW
_GC = 10
_NG = _NCH // _GC
_DW = 128
_NP = 10240
_RPT = _NP // _NS
_ZR = 32


def _scatter_sc(h):
    mesh = plsc.VectorSubcoreMesh(core_axis_name="c", subcore_axis_name="s")

    @functools.partial(
        pl.kernel,
        mesh=mesh,
        out_type=jax.ShapeDtypeStruct((_NC, _NP, h), jnp.float32),
        compiler_params=pltpu.CompilerParams(use_tc_tiling_on_sc=False),
        scratch_types=[
            pltpu.VMEM_SHARED((_NP, h), jnp.float32),
            pltpu.VMEM((_GC, 2, _CHUNK), jnp.int32),
            pltpu.VMEM((_GC, 2, _CHUNK), jnp.int32),
            pltpu.VMEM((_CHUNK, h), jnp.float32),
            pltpu.VMEM((_CHUNK, h), jnp.float32),
            pltpu.SemaphoreType.DMA,
            pltpu.SemaphoreType.DMA,
            pltpu.SemaphoreType.DMA,
        ],
    )
    def scat(y_hbm, e_hbm, out_hbm, acc_sh, ga, gb, rb0, rb1,
             gs0, gs1, isem):
        c = lax.axis_index("c")
        s = lax.axis_index("s")
        wid = c * _NS + s
        base = s * _RPT

        zv = jnp.zeros((16,), jnp.float32)

        def zrow(i, carry):
            def zcol(k, carry2):
                rb0[i, pl.ds(pl.multiple_of(k * 16, 16), 16)] = zv
                return carry2
            return lax.fori_loop(0, h // 16, zcol, carry)

        lax.fori_loop(0, _CHUNK, zrow, 0)
        for k in range(_RPT // _CHUNK):
            pltpu.sync_copy(rb0, acc_sh.at[pl.ds(base + k * _CHUNK, _CHUNK)])
        plsc.subcore_barrier()

        rbs = [rb0, rb1]
        sems = [gs0, gs1]

        def group(gbuf):
            hnd = [None] * _GC
            for j in range(2):
                hnd[j] = pltpu.async_copy(
                    y_hbm.at[gbuf.at[j, 0]], rbs[j % 2], sems[j % 2])
            for j in range(_GC):
                hnd[j].wait()
                pltpu.sync_copy(rbs[j % 2], acc_sh.at[gbuf.at[j, 1]],
                                add=True)
                if j + 2 < _GC:
                    hnd[j + 2] = pltpu.async_copy(
                        y_hbm.at[gbuf.at[j + 2, 0]], rbs[j % 2],
                        sems[j % 2])

        pltpu.sync_copy(e_hbm.at[wid, pl.ds(0, _GC)], ga)

        def pair(k, carry):
            g = 2 * k
            hb = pltpu.async_copy(
                e_hbm.at[wid, pl.ds(jnp.minimum(g + 1, _NG - 1) * _GC, _GC)],
                gb, isem)
            group(ga)
            hb.wait()
            ha = pltpu.async_copy(
                e_hbm.at[wid, pl.ds(jnp.minimum(g + 2, _NG - 1) * _GC, _GC)],
                ga, isem)
            group(gb)
            ha.wait()
            return carry

        lax.fori_loop(0, _NG // 2, pair, 0)
        plsc.subcore_barrier()

        pltpu.sync_copy(acc_sh.at[pl.ds(base, _RPT)],
                        out_hbm.at[c, pl.ds(base, _RPT)])

    return scat


def _degree_sc():
    mesh = plsc.VectorSubcoreMesh(core_axis_name="c", subcore_axis_name="s")
    h = _DW

    @functools.partial(
        pl.kernel,
        mesh=mesh,
        out_type=jax.ShapeDtypeStruct((_NC, _NP, h), jnp.float32),
        compiler_params=pltpu.CompilerParams(use_tc_tiling_on_sc=False),
        scratch_types=[
            pltpu.VMEM_SHARED((_NP, h), jnp.float32),
            pltpu.VMEM((_NCH, _CHUNK), jnp.int32),
            pltpu.VMEM((_CHUNK, h), jnp.float32),
            pltpu.VMEM((_CHUNK, h), jnp.float32),
        ],
    )
    def degk(dst_hbm, out_hbm, acc_sh, dst_v, ones_v, zbuf_v):
        c = lax.axis_index("c")
        s = lax.axis_index("s")
        wid = c * _NS + s
        base = s * _RPT

        pltpu.sync_copy(dst_hbm.at[wid], dst_v)

        zv = jnp.zeros((16,), jnp.float32)
        ov = jnp.ones((16,), jnp.float32)

        def fill(i, carry):
            def fcol(k, carry2):
                kk = pl.ds(pl.multiple_of(k * 16, 16), 16)
                zbuf_v[i, kk] = zv
                ones_v[i, kk] = ov
                return carry2
            return lax.fori_loop(0, h // 16, fcol, carry)

        lax.fori_loop(0, _CHUNK, fill, 0)
        for k in range(_RPT // _CHUNK):
            pltpu.sync_copy(zbuf_v, acc_sh.at[pl.ds(base + k * _CHUNK, _CHUNK)])
        plsc.subcore_barrier()

        def step(j, carry):
            pltpu.sync_copy(ones_v, acc_sh.at[dst_v.at[j]], add=True)
            return carry

        lax.fori_loop(0, _NCH, step, 0)
        plsc.subcore_barrier()

        pltpu.sync_copy(acc_sh.at[pl.ds(base, _RPT)],
                        out_hbm.at[c, pl.ds(base, _RPT)])

    return degk


def _mm_body(x_ref, w_ref, o_ref):
    o_ref[...] = jnp.dot(x_ref[...], w_ref[...],
                         preferred_element_type=jnp.float32)


def _k1_body(deg_ref, xw_ref, y_ref, dis_ref):
    deg = deg_ref[0, :_N, 0:1] + deg_ref[1, :_N, 0:1] + 1.0
    dis = lax.rsqrt(jnp.maximum(deg, 1e-12))
    y_ref[...] = xw_ref[...] * dis
    dis_ref[...] = dis


def _k2_body(hin, a_ref, y_ref, dis_ref, g_ref, be_ref, w_ref, o_ref):
    dis = dis_ref[...]
    v = (a_ref[0, :_N, :hin] + a_ref[1, :_N, :hin]
         + y_ref[..., :hin]) * dis
    mu = jnp.mean(v, axis=0, keepdims=True)
    var = jnp.mean((v - mu) * (v - mu), axis=0, keepdims=True)
    hbn = (v - mu) * lax.rsqrt(var + 1e-5) * g_ref[...] + be_ref[...]
    hr = jnp.maximum(hbn, 0.0)
    o_ref[...] = jnp.dot(hr, w_ref[...],
                         preferred_element_type=jnp.float32) * dis


def _k3_body(a_ref, y_ref, dis_ref, b_ref, batch_ref, o_ref):
    hfin = (a_ref[0, :_N] + a_ref[1, :_N] + y_ref[...]) * dis_ref[...] + b_ref[...]
    seg = lax.broadcasted_iota(jnp.int32, (_G, _N), 0)
    p = jnp.where(batch_ref[...] == seg, 1.0, 0.0)
    sums = jnp.dot(p, hfin, preferred_element_type=jnp.float32)
    cnt = jnp.sum(p, axis=1, keepdims=True)
    pooled = sums[:, :_C] / jnp.maximum(cnt, 1.0)
    m = jnp.max(pooled, axis=1, keepdims=True)
    ex = jnp.exp(pooled - m)
    lse = jnp.log(jnp.sum(ex, axis=1, keepdims=True)) + m
    o_ref[...] = pooled - lse


def _tc_call(body, out_shapes, *args):
    return pl.pallas_call(
        body,
        out_shape=out_shapes,
    )(*args)


def kernel(x, edge_index, batch, W1, b1, g1, be1, W2, b2, g2, be2, W3, b3):
    dum_src = jnp.broadcast_to(
        jnp.arange(_EPAD, dtype=jnp.int32)[None, :], (_NW, _EPAD))
    dum_dst = jnp.broadcast_to(
        (_N + jnp.arange(_EPAD, dtype=jnp.int32) % (_NP - _N))[None, :],
        (_NW, _EPAD))
    srcp = jnp.concatenate(
        [edge_index[0].reshape(_NW, _EPW), dum_src], axis=1)
    dstp = jnp.concatenate(
        [edge_index[1].reshape(_NW, _EPW), dum_dst], axis=1)
    edges = jnp.stack([srcp.reshape(_NW, _NCH, _CHUNK),
                       dstp.reshape(_NW, _NCH, _CHUNK)], axis=2)
    dst = dstp.reshape(_NW, _NCH, _CHUNK)
    batch2 = batch.reshape(1, _N)
    w3p = jnp.pad(W3, ((0, 0), (0, 16 - _C)))
    b3p = jnp.pad(b3, (0, 16 - _C)).reshape(1, 16)
    g1r = g1.reshape(1, -1)
    be1r = be1.reshape(1, -1)
    g2r = g2.reshape(1, -1)
    be2r = be2.reshape(1, -1)

    deg2 = _degree_sc()(dst)
    xw1 = _tc_call(
        _mm_body,
        jax.ShapeDtypeStruct((_N, 64), jnp.float32),
        x, W1)

    y1, dis = _tc_call(
        _k1_body,
        (jax.ShapeDtypeStruct((_N, 64), jnp.float32),
         jax.ShapeDtypeStruct((_N, 1), jnp.float32)),
        deg2, xw1)

    a1 = _scatter_sc(64)(y1, edges)
    y2 = _tc_call(
        functools.partial(_k2_body, 64),
        jax.ShapeDtypeStruct((_N, 128), jnp.float32),
        a1, y1, dis, g1r, be1r, W2)

    a2 = _scatter_sc(128)(y2, edges)
    y3 = _tc_call(
        functools.partial(_k2_body, 128),
        jax.ShapeDtypeStruct((_N, 16), jnp.float32),
        a2, y2, dis, g2r, be2r, w3p)

    a3 = _scatter_sc(16)(y3, edges)
    out = _tc_call(
        _k3_body,
        jax.ShapeDtypeStruct((_G, _C), jnp.float32),
        a3, y3, dis, b3p, batch2)
    return out


# speedup vs baseline: 33.4997x; 1.1204x over previous
import functools

import jax
import jax.numpy as jnp
from jax import lax
from jax.experimental import pallas as pl
from jax.experimental.pallas import tpu as pltpu
from jax.experimental.pallas import tpu_sc as plsc

_N = 10000
_E = 320000
_G = 64
_C = 10

_NC = 2
_NS = 16
_NW = _NC * _NS
_EPW = _E // _NW
_CHUNK = 128
_NCH = 80
_EPAD = _NCH * _CHUNK - _EPW
_GC = 10
_NG = _NCH // _GC
_DW = 16
_NP = 10240
_RPT = _NP // _NS
_ZR = 32


def _scatter_sc(h):
    mesh = plsc.VectorSubcoreMesh(core_axis_name="c", subcore_axis_name="s")

    @functools.partial(
        pl.kernel,
        mesh=mesh,
        out_type=jax.ShapeDtypeStruct((_NC, _NP, h), jnp.float32),
        compiler_params=pltpu.CompilerParams(use_tc_tiling_on_sc=False),
        scratch_types=[
            pltpu.VMEM_SHARED((_NP, h), jnp.float32),
            pltpu.VMEM((_GC, 2, _CHUNK), jnp.int32),
            pltpu.VMEM((_GC, 2, _CHUNK), jnp.int32),
            pltpu.VMEM((_CHUNK, h), jnp.float32),
            pltpu.VMEM((_CHUNK, h), jnp.float32),
            pltpu.SemaphoreType.DMA,
            pltpu.SemaphoreType.DMA,
            pltpu.SemaphoreType.DMA,
        ],
    )
    def scat(y_hbm, e_hbm, out_hbm, acc_sh, ga, gb, rb0, rb1,
             gs0, gs1, isem):
        c = lax.axis_index("c")
        s = lax.axis_index("s")
        wid = c * _NS + s
        base = s * _RPT

        zv = jnp.zeros((16,), jnp.float32)

        def zrow(i, carry):
            def zcol(k, carry2):
                rb0[i, pl.ds(pl.multiple_of(k * 16, 16), 16)] = zv
                return carry2
            return lax.fori_loop(0, h // 16, zcol, carry)

        lax.fori_loop(0, _CHUNK, zrow, 0)
        for k in range(_RPT // _CHUNK):
            pltpu.sync_copy(rb0, acc_sh.at[pl.ds(base + k * _CHUNK, _CHUNK)])
        plsc.subcore_barrier()

        rbs = [rb0, rb1]
        sems = [gs0, gs1]

        def group(gbuf):
            hnd = [None] * _GC
            for j in range(2):
                hnd[j] = pltpu.async_copy(
                    y_hbm.at[gbuf.at[j, 0]], rbs[j % 2], sems[j % 2])
            for j in range(_GC):
                hnd[j].wait()
                pltpu.sync_copy(rbs[j % 2], acc_sh.at[gbuf.at[j, 1]],
                                add=True)
                if j + 2 < _GC:
                    hnd[j + 2] = pltpu.async_copy(
                        y_hbm.at[gbuf.at[j + 2, 0]], rbs[j % 2],
                        sems[j % 2])

        pltpu.sync_copy(e_hbm.at[wid, pl.ds(0, _GC)], ga)

        def pair(k, carry):
            g = 2 * k
            hb = pltpu.async_copy(
                e_hbm.at[wid, pl.ds(jnp.minimum(g + 1, _NG - 1) * _GC, _GC)],
                gb, isem)
            group(ga)
            hb.wait()
            ha = pltpu.async_copy(
                e_hbm.at[wid, pl.ds(jnp.minimum(g + 2, _NG - 1) * _GC, _GC)],
                ga, isem)
            group(gb)
            ha.wait()
            return carry

        lax.fori_loop(0, _NG // 2, pair, 0)
        plsc.subcore_barrier()

        pltpu.sync_copy(acc_sh.at[pl.ds(base, _RPT)],
                        out_hbm.at[c, pl.ds(base, _RPT)])

    return scat


def _degree_sc():
    mesh = plsc.VectorSubcoreMesh(core_axis_name="c", subcore_axis_name="s")
    h = _DW

    @functools.partial(
        pl.kernel,
        mesh=mesh,
        out_type=jax.ShapeDtypeStruct((_NC, _NP, h), jnp.float32),
        compiler_params=pltpu.CompilerParams(use_tc_tiling_on_sc=False),
        scratch_types=[
            pltpu.VMEM_SHARED((_NP, h), jnp.float32),
            pltpu.VMEM((_NCH, _CHUNK), jnp.int32),
            pltpu.VMEM((_CHUNK, h), jnp.float32),
            pltpu.VMEM((_CHUNK, h), jnp.float32),
        ],
    )
    def degk(dst_hbm, out_hbm, acc_sh, dst_v, ones_v, zbuf_v):
        c = lax.axis_index("c")
        s = lax.axis_index("s")
        wid = c * _NS + s
        base = s * _RPT

        pltpu.sync_copy(dst_hbm.at[wid], dst_v)

        zv = jnp.zeros((16,), jnp.float32)
        ov = jnp.ones((16,), jnp.float32)

        def fill(i, carry):
            def fcol(k, carry2):
                kk = pl.ds(pl.multiple_of(k * 16, 16), 16)
                zbuf_v[i, kk] = zv
                ones_v[i, kk] = ov
                return carry2
            return lax.fori_loop(0, h // 16, fcol, carry)

        lax.fori_loop(0, _CHUNK, fill, 0)
        for k in range(_RPT // _CHUNK):
            pltpu.sync_copy(zbuf_v, acc_sh.at[pl.ds(base + k * _CHUNK, _CHUNK)])
        plsc.subcore_barrier()

        def step(j, carry):
            pltpu.sync_copy(ones_v, acc_sh.at[dst_v.at[j]], add=True)
            return carry

        lax.fori_loop(0, _NCH, step, 0)
        plsc.subcore_barrier()

        pltpu.sync_copy(acc_sh.at[pl.ds(base, _RPT)],
                        out_hbm.at[c, pl.ds(base, _RPT)])

    return degk


def _mm_body(x_ref, w_ref, o_ref):
    o_ref[...] = jnp.dot(x_ref[...], w_ref[...],
                         preferred_element_type=jnp.float32)


def _k1_body(deg_ref, xw_ref, y_ref, dis_ref):
    deg = deg_ref[0, :_N, 0:1] + deg_ref[1, :_N, 0:1] + 1.0
    dis = lax.rsqrt(jnp.maximum(deg, 1e-12))
    y_ref[...] = xw_ref[...] * dis
    dis_ref[...] = dis


def _k2_body(hin, a_ref, y_ref, dis_ref, g_ref, be_ref, w_ref, o_ref):
    dis = dis_ref[...]
    v = (a_ref[0, :_N, :hin] + a_ref[1, :_N, :hin]
         + y_ref[..., :hin]) * dis
    mu = jnp.mean(v, axis=0, keepdims=True)
    var = jnp.mean((v - mu) * (v - mu), axis=0, keepdims=True)
    hbn = (v - mu) * lax.rsqrt(var + 1e-5) * g_ref[...] + be_ref[...]
    hr = jnp.maximum(hbn, 0.0)
    o_ref[...] = jnp.dot(hr, w_ref[...],
                         preferred_element_type=jnp.float32) * dis


def _k3_body(a_ref, y_ref, dis_ref, b_ref, batch_ref, o_ref):
    hfin = (a_ref[0, :_N] + a_ref[1, :_N] + y_ref[...]) * dis_ref[...] + b_ref[...]
    seg = lax.broadcasted_iota(jnp.int32, (_G, _N), 0)
    p = jnp.where(batch_ref[...] == seg, 1.0, 0.0)
    sums = jnp.dot(p, hfin, preferred_element_type=jnp.float32)
    cnt = jnp.sum(p, axis=1, keepdims=True)
    pooled = sums[:, :_C] / jnp.maximum(cnt, 1.0)
    m = jnp.max(pooled, axis=1, keepdims=True)
    ex = jnp.exp(pooled - m)
    lse = jnp.log(jnp.sum(ex, axis=1, keepdims=True)) + m
    o_ref[...] = pooled - lse


def _tc_call(body, out_shapes, *args):
    return pl.pallas_call(
        body,
        out_shape=out_shapes,
    )(*args)


def kernel(x, edge_index, batch, W1, b1, g1, be1, W2, b2, g2, be2, W3, b3):
    dum_src = jnp.broadcast_to(
        jnp.arange(_EPAD, dtype=jnp.int32)[None, :], (_NW, _EPAD))
    dum_dst = jnp.broadcast_to(
        (_N + jnp.arange(_EPAD, dtype=jnp.int32) % (_NP - _N))[None, :],
        (_NW, _EPAD))
    srcp = jnp.concatenate(
        [edge_index[0].reshape(_NW, _EPW), dum_src], axis=1)
    dstp = jnp.concatenate(
        [edge_index[1].reshape(_NW, _EPW), dum_dst], axis=1)
    edges = jnp.stack([srcp.reshape(_NW, _NCH, _CHUNK),
                       dstp.reshape(_NW, _NCH, _CHUNK)], axis=2)
    dst = dstp.reshape(_NW, _NCH, _CHUNK)
    batch2 = batch.reshape(1, _N)
    w3p = jnp.pad(W3, ((0, 0), (0, 16 - _C)))
    b3p = jnp.pad(b3, (0, 16 - _C)).reshape(1, 16)
    g1r = g1.reshape(1, -1)
    be1r = be1.reshape(1, -1)
    g2r = g2.reshape(1, -1)
    be2r = be2.reshape(1, -1)

    deg2 = _degree_sc()(dst)
    xw1 = _tc_call(
        _mm_body,
        jax.ShapeDtypeStruct((_N, 64), jnp.float32),
        x, W1)

    y1, dis = _tc_call(
        _k1_body,
        (jax.ShapeDtypeStruct((_N, 64), jnp.float32),
         jax.ShapeDtypeStruct((_N, 1), jnp.float32)),
        deg2, xw1)

    a1 = _scatter_sc(64)(y1, edges)
    y2 = _tc_call(
        functools.partial(_k2_body, 64),
        jax.ShapeDtypeStruct((_N, 128), jnp.float32),
        a1, y1, dis, g1r, be1r, W2)

    a2 = _scatter_sc(128)(y2, edges)
    y3 = _tc_call(
        functools.partial(_k2_body, 128),
        jax.ShapeDtypeStruct((_N, 16), jnp.float32),
        a2, y2, dis, g2r, be2r, w3p)

    a3 = _scatter_sc(16)(y3, edges)
    out = _tc_call(
        _k3_body,
        jax.ShapeDtypeStruct((_G, _C), jnp.float32),
        a3, y3, dis, b3p, batch2)
    return out


# speedup vs baseline: 34.3186x; 1.0244x over previous
import functools

import jax
import jax.numpy as jnp
from jax import lax
from jax.experimental import pallas as pl
from jax.experimental.pallas import tpu as pltpu
from jax.experimental.pallas import tpu_sc as plsc

_N = 10000
_E = 320000
_G = 64
_C = 10

_NC = 2
_NS = 16
_NW = _NC * _NS
_EPW = _E // _NW
_CHUNK = 128
_NCH = 80
_EPAD = _NCH * _CHUNK - _EPW
_GC = 20
_NG = _NCH // _GC
_DW = 16
_NP = 10240
_RPT = _NP // _NS
_ZR = 32


def _scatter_sc(h):
    mesh = plsc.VectorSubcoreMesh(core_axis_name="c", subcore_axis_name="s")

    @functools.partial(
        pl.kernel,
        mesh=mesh,
        out_type=jax.ShapeDtypeStruct((_NC, _NP, h), jnp.float32),
        compiler_params=pltpu.CompilerParams(use_tc_tiling_on_sc=False),
        scratch_types=[
            pltpu.VMEM_SHARED((_NP, h), jnp.float32),
            pltpu.VMEM((_GC, 2, _CHUNK), jnp.int32),
            pltpu.VMEM((_GC, 2, _CHUNK), jnp.int32),
            pltpu.VMEM((_CHUNK, h), jnp.float32),
            pltpu.VMEM((_CHUNK, h), jnp.float32),
            pltpu.SemaphoreType.DMA,
            pltpu.SemaphoreType.DMA,
            pltpu.SemaphoreType.DMA,
        ],
    )
    def scat(y_hbm, e_hbm, out_hbm, acc_sh, ga, gb, rb0, rb1,
             gs0, gs1, isem):
        c = lax.axis_index("c")
        s = lax.axis_index("s")
        wid = c * _NS + s
        base = s * _RPT

        zv = jnp.zeros((16,), jnp.float32)

        def zrow(i, carry):
            def zcol(k, carry2):
                rb0[i, pl.ds(pl.multiple_of(k * 16, 16), 16)] = zv
                return carry2
            return lax.fori_loop(0, h // 16, zcol, carry)

        lax.fori_loop(0, _CHUNK, zrow, 0)
        for k in range(_RPT // _CHUNK):
            pltpu.sync_copy(rb0, acc_sh.at[pl.ds(base + k * _CHUNK, _CHUNK)])
        plsc.subcore_barrier()

        rbs = [rb0, rb1]
        sems = [gs0, gs1]

        def group(gbuf):
            hnd = [None] * _GC
            for j in range(2):
                hnd[j] = pltpu.async_copy(
                    y_hbm.at[gbuf.at[j, 0]], rbs[j % 2], sems[j % 2])
            for j in range(_GC):
                hnd[j].wait()
                pltpu.sync_copy(rbs[j % 2], acc_sh.at[gbuf.at[j, 1]],
                                add=True)
                if j + 2 < _GC:
                    hnd[j + 2] = pltpu.async_copy(
                        y_hbm.at[gbuf.at[j + 2, 0]], rbs[j % 2],
                        sems[j % 2])

        pltpu.sync_copy(e_hbm.at[wid, pl.ds(0, _GC)], ga)

        def pair(k, carry):
            g = 2 * k
            hb = pltpu.async_copy(
                e_hbm.at[wid, pl.ds(jnp.minimum(g + 1, _NG - 1) * _GC, _GC)],
                gb, isem)
            group(ga)
            hb.wait()
            ha = pltpu.async_copy(
                e_hbm.at[wid, pl.ds(jnp.minimum(g + 2, _NG - 1) * _GC, _GC)],
                ga, isem)
            group(gb)
            ha.wait()
            return carry

        lax.fori_loop(0, _NG // 2, pair, 0)
        plsc.subcore_barrier()

        pltpu.sync_copy(acc_sh.at[pl.ds(base, _RPT)],
                        out_hbm.at[c, pl.ds(base, _RPT)])

    return scat


def _degree_sc():
    mesh = plsc.VectorSubcoreMesh(core_axis_name="c", subcore_axis_name="s")
    h = _DW

    @functools.partial(
        pl.kernel,
        mesh=mesh,
        out_type=jax.ShapeDtypeStruct((_NC, _NP, h), jnp.float32),
        compiler_params=pltpu.CompilerParams(use_tc_tiling_on_sc=False),
        scratch_types=[
            pltpu.VMEM_SHARED((_NP, h), jnp.float32),
            pltpu.VMEM((_NCH, _CHUNK), jnp.int32),
            pltpu.VMEM((_CHUNK, h), jnp.float32),
            pltpu.VMEM((_CHUNK, h), jnp.float32),
        ],
    )
    def degk(dst_hbm, out_hbm, acc_sh, dst_v, ones_v, zbuf_v):
        c = lax.axis_index("c")
        s = lax.axis_index("s")
        wid = c * _NS + s
        base = s * _RPT

        pltpu.sync_copy(dst_hbm.at[wid], dst_v)

        zv = jnp.zeros((16,), jnp.float32)
        ov = jnp.ones((16,), jnp.float32)

        def fill(i, carry):
            def fcol(k, carry2):
                kk = pl.ds(pl.multiple_of(k * 16, 16), 16)
                zbuf_v[i, kk] = zv
                ones_v[i, kk] = ov
                return carry2
            return lax.fori_loop(0, h // 16, fcol, carry)

        lax.fori_loop(0, _CHUNK, fill, 0)
        for k in range(_RPT // _CHUNK):
            pltpu.sync_copy(zbuf_v, acc_sh.at[pl.ds(base + k * _CHUNK, _CHUNK)])
        plsc.subcore_barrier()

        def step(j, carry):
            pltpu.sync_copy(ones_v, acc_sh.at[dst_v.at[j]], add=True)
            return carry

        lax.fori_loop(0, _NCH, step, 0)
        plsc.subcore_barrier()

        pltpu.sync_copy(acc_sh.at[pl.ds(base, _RPT)],
                        out_hbm.at[c, pl.ds(base, _RPT)])

    return degk


def _mm_body(x_ref, w_ref, o_ref):
    o_ref[...] = jnp.dot(x_ref[...], w_ref[...],
                         preferred_element_type=jnp.float32)


def _k1_body(deg_ref, xw_ref, y_ref, dis_ref):
    deg = deg_ref[0, :_N, 0:1] + deg_ref[1, :_N, 0:1] + 1.0
    dis = lax.rsqrt(jnp.maximum(deg, 1e-12))
    y_ref[...] = xw_ref[...] * dis
    dis_ref[...] = dis


def _k2_body(hin, a_ref, y_ref, dis_ref, g_ref, be_ref, w_ref, o_ref):
    dis = dis_ref[...]
    v = (a_ref[0, :_N, :hin] + a_ref[1, :_N, :hin]
         + y_ref[..., :hin]) * dis
    mu = jnp.mean(v, axis=0, keepdims=True)
    var = jnp.mean((v - mu) * (v - mu), axis=0, keepdims=True)
    hbn = (v - mu) * lax.rsqrt(var + 1e-5) * g_ref[...] + be_ref[...]
    hr = jnp.maximum(hbn, 0.0)
    o_ref[...] = jnp.dot(hr, w_ref[...],
                         preferred_element_type=jnp.float32) * dis


def _k3_body(a_ref, y_ref, dis_ref, b_ref, batch_ref, o_ref):
    hfin = (a_ref[0, :_N] + a_ref[1, :_N] + y_ref[...]) * dis_ref[...] + b_ref[...]
    seg = lax.broadcasted_iota(jnp.int32, (_G, _N), 0)
    p = jnp.where(batch_ref[...] == seg, 1.0, 0.0)
    sums = jnp.dot(p, hfin, preferred_element_type=jnp.float32)
    cnt = jnp.sum(p, axis=1, keepdims=True)
    pooled = sums[:, :_C] / jnp.maximum(cnt, 1.0)
    m = jnp.max(pooled, axis=1, keepdims=True)
    ex = jnp.exp(pooled - m)
    lse = jnp.log(jnp.sum(ex, axis=1, keepdims=True)) + m
    o_ref[...] = pooled - lse


def _tc_call(body, out_shapes, *args):
    return pl.pallas_call(
        body,
        out_shape=out_shapes,
    )(*args)


def kernel(x, edge_index, batch, W1, b1, g1, be1, W2, b2, g2, be2, W3, b3):
    dum_src = jnp.broadcast_to(
        jnp.arange(_EPAD, dtype=jnp.int32)[None, :], (_NW, _EPAD))
    dum_dst = jnp.broadcast_to(
        (_N + jnp.arange(_EPAD, dtype=jnp.int32) % (_NP - _N))[None, :],
        (_NW, _EPAD))
    srcp = jnp.concatenate(
        [edge_index[0].reshape(_NW, _EPW), dum_src], axis=1)
    dstp = jnp.concatenate(
        [edge_index[1].reshape(_NW, _EPW), dum_dst], axis=1)
    edges = jnp.stack([srcp.reshape(_NW, _NCH, _CHUNK),
                       dstp.reshape(_NW, _NCH, _CHUNK)], axis=2)
    dst = dstp.reshape(_NW, _NCH, _CHUNK)
    batch2 = batch.reshape(1, _N)
    w3p = jnp.pad(W3, ((0, 0), (0, 16 - _C)))
    b3p = jnp.pad(b3, (0, 16 - _C)).reshape(1, 16)
    g1r = g1.reshape(1, -1)
    be1r = be1.reshape(1, -1)
    g2r = g2.reshape(1, -1)
    be2r = be2.reshape(1, -1)

    deg2 = _degree_sc()(dst)
    xw1 = _tc_call(
        _mm_body,
        jax.ShapeDtypeStruct((_N, 64), jnp.float32),
        x, W1)

    y1, dis = _tc_call(
        _k1_body,
        (jax.ShapeDtypeStruct((_N, 64), jnp.float32),
         jax.ShapeDtypeStruct((_N, 1), jnp.float32)),
        deg2, xw1)

    a1 = _scatter_sc(64)(y1, edges)
    y2 = _tc_call(
        functools.partial(_k2_body, 64),
        jax.ShapeDtypeStruct((_N, 128), jnp.float32),
        a1, y1, dis, g1r, be1r, W2)

    a2 = _scatter_sc(128)(y2, edges)
    y3 = _tc_call(
        functools.partial(_k2_body, 128),
        jax.ShapeDtypeStruct((_N, 16), jnp.float32),
        a2, y2, dis, g2r, be2r, w3p)

    a3 = _scatter_sc(16)(y3, edges)
    out = _tc_call(
        _k3_body,
        jax.ShapeDtypeStruct((_G, _C), jnp.float32),
        a3, y3, dis, b3p, batch2)
    return out
